# baseline (speedup 1.0000x reference)
import jax
import jax.numpy as jnp
from jax.experimental import pallas as pl
from jax.experimental.pallas import tpu as pltpu

_ROWS = 8192
_COLS = 1024
_NCHUNK = 8
_CHUNK = _ROWS // _NCHUNK


def _copy_body(w_ref, o_ref, *sems):
    for c in range(_NCHUNK):
        pltpu.make_async_copy(
            w_ref.at[pl.ds(c * _CHUNK, _CHUNK), :],
            o_ref.at[pl.ds(c * _CHUNK, _CHUNK), :],
            sems[c],
        ).start()
    for c in range(_NCHUNK):
        pltpu.make_async_copy(
            w_ref.at[pl.ds(c * _CHUNK, _CHUNK), :],
            o_ref.at[pl.ds(c * _CHUNK, _CHUNK), :],
            sems[c],
        ).wait()


def kernel(x, W):
    del x
    return pl.pallas_call(
        _copy_body,
        in_specs=[pl.BlockSpec(memory_space=pl.ANY)],
        out_specs=pl.BlockSpec(memory_space=pl.ANY),
        out_shape=jax.ShapeDtypeStruct((_ROWS, _COLS), W.dtype),
        scratch_shapes=[pltpu.SemaphoreType.DMA] * _NCHUNK,
    )(W)


# speedup vs baseline: 24.4577x; 24.4577x over previous
import jax
import jax.numpy as jnp
from jax import lax
from jax.experimental import pallas as pl
from jax.experimental.pallas import tpu as pltpu
from jax.experimental.pallas import tpu_sc as plsc

_ROWS = 8192
_COLS = 1024
_NC = 2
_NS = 16
_NW = _NC * _NS
_ROWS_PER_W = _ROWS // _NW
_BLK = 32
_NBLK = _ROWS_PER_W // _BLK
_NBUF = 3


def _sc_copy_body(w_hbm, o_hbm, *scratch):
    bufs = scratch[:_NBUF]
    sin = scratch[_NBUF:2 * _NBUF]
    sout = scratch[2 * _NBUF:3 * _NBUF]
    wid = lax.axis_index("s") * _NC + lax.axis_index("c")
    base = wid * _ROWS_PER_W

    def in_copy(i, b):
        return pltpu.make_async_copy(
            w_hbm.at[pl.ds(base + i * _BLK, _BLK), :], bufs[b], sin[b])

    def out_copy(i, b):
        return pltpu.make_async_copy(
            bufs[b], o_hbm.at[pl.ds(base + i * _BLK, _BLK), :], sout[b])

    for i in range(min(_NBUF, _NBLK)):
        in_copy(i, i % _NBUF).start()
    for i in range(_NBLK):
        b = i % _NBUF
        in_copy(i, b).wait()
        out_copy(i, b).start()
        nxt = i + _NBUF
        if nxt < _NBLK:
            out_copy(i, b).wait()
            in_copy(nxt, b).start()
    for i in range(max(0, _NBLK - _NBUF), _NBLK):
        out_copy(i, i % _NBUF).wait()


def kernel(x, W):
    del x
    mesh = plsc.VectorSubcoreMesh(core_axis_name="c", subcore_axis_name="s")
    scratch = (
        [pltpu.VMEM((_BLK, _COLS), jnp.float32)] * _NBUF
        + [pltpu.SemaphoreType.DMA] * (2 * _NBUF)
    )
    f = pl.kernel(
        _sc_copy_body,
        out_type=jax.ShapeDtypeStruct((_ROWS, _COLS), W.dtype),
        mesh=mesh,
        scratch_types=scratch,
    )
    return f(W)
